# baseline (speedup 1.0000x reference)
import functools

import jax
import jax.numpy as jnp
from jax import lax
from jax.experimental import pallas as pl
from jax.experimental.pallas import tpu as pltpu
from jax.experimental.pallas import tpu_sc as plsc

N_ATOMS = 500000
NALTERN = 4
NBINS = 8 * 4 * 512
ACC_WORDS = NBINS * NALTERN
N_SUBCORES = 16
N_CORES = 2

BLK_ATOMS = 4000
N_BLKS = N_ATOMS // BLK_ATOMS
BLK_OUT = 4096
PAD_OUT = BLK_OUT - BLK_ATOMS
STREAM = N_BLKS * BLK_OUT

ATOMS_PER_SUB = STREAM // N_SUBCORES
SC_CHUNK = 4000
SC_NCHUNK = ATOMS_PER_SUB // SC_CHUNK
SC_UNROLL = 2
SC_STEPS = SC_CHUNK // (16 * SC_UNROLL)


def _tc_prep_kernel(desc_ref, mask_ref, facc_ref, props_ref,
                    sb_ref, v0_ref, v1_ref, v2_ref, v3_ref):
    d = desc_ref[...]
    b = d[:, 0:1]
    ch = d[:, 1:2]
    r = d[:, 2:3]
    at = d[:, 3:4]
    cls = (at >= 4).astype(jnp.int32)
    base = cls * 65536 + b * 8192 + ch * 2048 + r * 4
    onehot = (at == lax.broadcasted_iota(jnp.int32, (1, 40), 1)).astype(jnp.float32)
    vdw = jnp.dot(onehot, props_ref[:, 0:1], precision=lax.Precision.HIGHEST)
    va = jnp.where(mask_ref[...], jnp.maximum(facc_ref[...], 0.0) * vdw, 0.0)
    zi = jnp.zeros((PAD_OUT,), jnp.int32)
    zf = jnp.zeros((PAD_OUT,), jnp.float32)
    sb_ref[...] = jnp.concatenate([jnp.squeeze(base, axis=1), zi])
    for a, ref in enumerate((v0_ref, v1_ref, v2_ref, v3_ref)):
        ref[...] = jnp.concatenate([va[:, a], zf])


def _tc_prep(atom_description, alternativeMask, facc, atom_Properties):
    fstream = jax.ShapeDtypeStruct((STREAM,), jnp.float32)
    oblock = pl.BlockSpec((BLK_OUT,), lambda i: (i,))
    return pl.pallas_call(
        _tc_prep_kernel,
        grid=(N_BLKS,),
        in_specs=[
            pl.BlockSpec((BLK_ATOMS, 4), lambda i: (i, 0)),
            pl.BlockSpec((BLK_ATOMS, NALTERN), lambda i: (i, 0)),
            pl.BlockSpec((BLK_ATOMS, NALTERN), lambda i: (i, 0)),
            pl.BlockSpec((40, 8), lambda i: (0, 0)),
        ],
        out_specs=[oblock] * 5,
        out_shape=[jax.ShapeDtypeStruct((STREAM,), jnp.int32)] + [fstream] * 4,
    )(atom_description, alternativeMask, facc, atom_Properties)


def _sc_scatter_kernel(sb_hbm, v0_hbm, v1_hbm, v2_hbm, v3_hbm, out_hbm,
                       sb_v, v0_v, v1_v, v2_v, v3_v, acc_v):
    c = lax.axis_index("c")
    s = lax.axis_index("s")

    zeros16 = jnp.zeros((16,), jnp.float32)

    def zero_body(i, carry):
        acc_v[pl.ds(i * 16, 16)] = zeros16
        return carry

    lax.fori_loop(0, ACC_WORDS // 16, zero_body, 0)

    cvec = jnp.broadcast_to(c, (16,)).astype(jnp.int32)
    sub_base = s * ATOMS_PER_SUB
    vrefs = (v0_v, v1_v, v2_v, v3_v)

    def chunk_body(k, carry):
        base = sub_base + k * SC_CHUNK
        pltpu.sync_copy(sb_hbm.at[pl.ds(base, SC_CHUNK)], sb_v)
        pltpu.sync_copy(v0_hbm.at[pl.ds(base, SC_CHUNK)], v0_v)
        pltpu.sync_copy(v1_hbm.at[pl.ds(base, SC_CHUNK)], v1_v)
        pltpu.sync_copy(v2_hbm.at[pl.ds(base, SC_CHUNK)], v2_v)
        pltpu.sync_copy(v3_hbm.at[pl.ds(base, SC_CHUNK)], v3_v)

        def step_body(t, carry2):
            o = t * (16 * SC_UNROLL)
            for u in range(SC_UNROLL):
                w = sb_v[pl.ds(o + u * 16, 16)]
                sel = (w >> 16) == cvec
                cell = w & 65535
                for a in range(NALTERN):
                    v = vrefs[a][pl.ds(o + u * 16, 16)]
                    plsc.addupdate_scatter(acc_v, [cell + a], v, mask=sel)
            return carry2

        lax.fori_loop(0, SC_STEPS, step_body, 0)
        return carry

    lax.fori_loop(0, SC_NCHUNK, chunk_body, 0)

    pltpu.sync_copy(acc_v, out_hbm.at[c, s])


_sc_scatter = functools.partial(
    pl.kernel,
    out_type=jax.ShapeDtypeStruct((N_CORES, N_SUBCORES, ACC_WORDS), jnp.float32),
    mesh=plsc.VectorSubcoreMesh(core_axis_name="c", subcore_axis_name="s"),
    compiler_params=pltpu.CompilerParams(needs_layout_passes=False),
    scratch_types=[
        pltpu.VMEM((SC_CHUNK,), jnp.int32),
        pltpu.VMEM((SC_CHUNK,), jnp.float32),
        pltpu.VMEM((SC_CHUNK,), jnp.float32),
        pltpu.VMEM((SC_CHUNK,), jnp.float32),
        pltpu.VMEM((SC_CHUNK,), jnp.float32),
        pltpu.VMEM((ACC_WORDS,), jnp.float32),
    ],
)(_sc_scatter_kernel)


def _tc_reduce_kernel(p_ref, w_ref, out_ref):
    scale = (1.0 - jnp.tanh(w_ref[0, 0])) * 0.3
    out_ref[...] = jnp.sum(p_ref[...], axis=1) * scale


def _tc_reduce(partials, weight):
    cols = ACC_WORDS // 8
    return pl.pallas_call(
        _tc_reduce_kernel,
        grid=(8,),
        in_specs=[
            pl.BlockSpec((N_CORES, N_SUBCORES, cols), lambda j: (0, 0, j)),
            pl.BlockSpec(memory_space=pltpu.SMEM),
        ],
        out_specs=pl.BlockSpec((N_CORES, cols), lambda j: (0, j)),
        out_shape=jax.ShapeDtypeStruct((N_CORES, ACC_WORDS), jnp.float32),
    )(partials, weight)


@jax.jit
def kernel(coords, atom_description, alternativeMask, facc, weight, atom_Properties):
    del coords
    sb, v0, v1, v2, v3 = _tc_prep(atom_description, alternativeMask, facc,
                                  atom_Properties)
    partials = _sc_scatter(sb, v0, v1, v2, v3)
    out2 = _tc_reduce(partials, weight.reshape(1, 1))
    final_mc = out2[0].reshape(8, 4, 512, NALTERN)
    final_sc = out2[1].reshape(8, 4, 512, NALTERN)
    return (final_mc, final_sc)
